# Initial kernel scaffold; baseline (speedup 1.0000x reference)
import jax
import jax.numpy as jnp
from jax.experimental import pallas as pl


def kernel(query, key, value, index_pair, query_batch_cnt, key_batch_cnt, index_pair_batch, relative_atten_weights, rpe_distance, Wq, bq, Wk, bk, Wv, bv, Wg1, bg1, Wg2, bg2):
    raise NotImplementedError("write your pallas kernel here")


# speedup vs baseline: 2.2467x; 2.2467x over previous
import functools

import jax
import jax.numpy as jnp
from jax import lax
from jax.experimental import pallas as pl
from jax.experimental.pallas import tpu as pltpu
from jax.experimental.pallas import tpu_sc as plsc

P, NBR, D, H = 8192, 16, 512, 16
HD = D // H
G = H * 3
GP = 128
R = P * NBR


BLK1 = 256


def _proj_kernel(xq_ref, xk_ref, xv_ref, wq_ref, bq_ref, wk_ref, bk_ref,
                 wv_ref, bv_ref, wg1_ref, bg1_ref,
                 q_ref, k_ref, v_ref, a_ref, bqo_ref):
    q = jnp.dot(xq_ref[...], wq_ref[...], preferred_element_type=jnp.float32) + bq_ref[...]
    k = jnp.dot(xk_ref[...], wk_ref[...], preferred_element_type=jnp.float32) + bk_ref[...]
    v = jnp.dot(xv_ref[...], wv_ref[...], preferred_element_type=jnp.float32) + bv_ref[...]
    q_ref[...] = q
    k_ref[...] = k
    v_ref[...] = v
    a_ref[...] = jnp.dot(k, wg1_ref[...], preferred_element_type=jnp.float32)
    bqo_ref[...] = jnp.dot(q, wg1_ref[...], preferred_element_type=jnp.float32) - bg1_ref[...]


def _stage1(query, key, value, Wq, bq, Wk, bk, Wv, bv, Wg1, bg1):
    n_blk = P // BLK1
    row_spec = pl.BlockSpec((BLK1, D), lambda i: (i, 0))
    w_spec = pl.BlockSpec((D, D), lambda i: (0, 0))
    b_spec = pl.BlockSpec((1, D), lambda i: (0, 0))
    g_spec = pl.BlockSpec((D, GP), lambda i: (0, 0))
    gb_spec = pl.BlockSpec((1, GP), lambda i: (0, 0))
    out_row = pl.BlockSpec((BLK1, D), lambda i: (i, 0))
    out_g = pl.BlockSpec((BLK1, GP), lambda i: (i, 0))
    return pl.pallas_call(
        _proj_kernel,
        grid=(n_blk,),
        in_specs=[row_spec, row_spec, row_spec, w_spec, b_spec, w_spec,
                  b_spec, w_spec, b_spec, g_spec, gb_spec],
        out_specs=[out_row, out_row, out_row, out_g, out_g],
        out_shape=[
            jax.ShapeDtypeStruct((P, D), jnp.float32),
            jax.ShapeDtypeStruct((P, D), jnp.float32),
            jax.ShapeDtypeStruct((P, D), jnp.float32),
            jax.ShapeDtypeStruct((P, GP), jnp.float32),
            jax.ShapeDtypeStruct((P, GP), jnp.float32),
        ],
    )(query, key, value, Wq, bq.reshape(1, D), Wk, bk.reshape(1, D),
      Wv, bv.reshape(1, D), Wg1, bg1.reshape(1, GP))


NW = 32
PAIRS_W = R // NW
CH = 64
NCH = PAIRS_W // CH


def _sc_gather(k_hbm, v_hbm, a_hbm, idx_hbm, kg_hbm, vg_hbm, ag_hbm,
               idxv, kbuf, vbuf, abuf, semk, semv, sema):
    wid = lax.axis_index("s") * 2 + lax.axis_index("c")
    base = wid * PAIRS_W

    @pl.loop(0, NCH)
    def _(ch):
        off = base + ch * CH
        pltpu.sync_copy(idx_hbm.at[pl.ds(off, CH)], idxv)
        ck = pltpu.async_copy(k_hbm.at[idxv], kbuf, semk)
        cv = pltpu.async_copy(v_hbm.at[idxv], vbuf, semv)
        ca = pltpu.async_copy(a_hbm.at[idxv], abuf, sema)
        ck.wait()
        cv.wait()
        ca.wait()
        pltpu.sync_copy(kbuf, kg_hbm.at[pl.ds(off, CH)])
        pltpu.sync_copy(vbuf, vg_hbm.at[pl.ds(off, CH)])
        pltpu.sync_copy(abuf, ag_hbm.at[pl.ds(off, CH)])


def _stage2(k, v, a, idx_flat):
    mesh = plsc.VectorSubcoreMesh(core_axis_name="c", subcore_axis_name="s")
    kern = functools.partial(
        pl.kernel,
        out_type=(
            jax.ShapeDtypeStruct((R, D), jnp.float32),
            jax.ShapeDtypeStruct((R, D), jnp.float32),
            jax.ShapeDtypeStruct((R, GP), jnp.float32),
        ),
        mesh=mesh,
        scratch_types=[
            pltpu.VMEM((CH,), jnp.int32),
            pltpu.VMEM((CH, D), jnp.float32),
            pltpu.VMEM((CH, D), jnp.float32),
            pltpu.VMEM((CH, GP), jnp.float32),
            pltpu.SemaphoreType.DMA,
            pltpu.SemaphoreType.DMA,
            pltpu.SemaphoreType.DMA,
        ],
    )(_sc_gather)
    return kern(k, v, a, idx_flat)


BLK3 = 128
RB = BLK3 * NBR


def _attn_kernel(q_ref, bq_ref, kg_ref, vg_ref, ag_ref, rpe_ref, rel_ref,
                 wg2_ref, bg2_ref, md_ref, m48_ref, t3_ref, e_ref, out_ref):
    q = q_ref[...]
    kg = kg_ref[...]
    vg = vg_ref[...]

    qexp = jnp.broadcast_to(q[:, None, :], (BLK3, NBR, D)).reshape(RB, D)
    dot = jnp.dot(qexp * kg, md_ref[...], preferred_element_type=jnp.float32)

    bq = bq_ref[...]
    bqexp = jnp.broadcast_to(bq[:, None, :], (BLK3, NBR, GP)).reshape(RB, GP)
    pre = jnp.maximum(ag_ref[...] - bqexp, 0.0)
    t = jnp.dot(pre, wg2_ref[...], preferred_element_type=jnp.float32) + bg2_ref[...]

    rpe = rpe_ref[...]
    n2 = jnp.sum(rpe * rpe, axis=1, keepdims=True)
    ln = jnp.sqrt(n2)
    u = rpe / jnp.maximum(ln, 1e-12)
    a_c, b_c = 0.001, 0.005
    ramp = 0.5 * (1.0 - jnp.cos(jnp.pi * (ln - a_c) / (b_c - a_c)))
    cut = jnp.where(ln < a_c, 0.0, jnp.where(ln > b_c, 1.0, ramp))
    shc = jnp.sqrt(3.0) * u * cut
    sht = jnp.dot(shc, t3_ref[...], preferred_element_type=jnp.float32)

    pos = jnp.dot(t * sht, m48_ref[...], preferred_element_type=jnp.float32)

    logits = (dot + pos + rel_ref[...]) * (1.0 / jnp.sqrt(jnp.float32(HD)))
    lg = logits.reshape(BLK3, NBR, H)
    m = jnp.max(lg, axis=1, keepdims=True)
    e = jnp.exp(lg - m)
    w = e / jnp.sum(e, axis=1, keepdims=True)
    wexp = jnp.dot(w.reshape(RB, H), e_ref[...], preferred_element_type=jnp.float32)
    out_ref[...] = jnp.sum((wexp * vg).reshape(BLK3, NBR, D), axis=1)


def _stage3(q, bqv, kg, vg, ag, rpe_flat, rel_flat, Wg2, bg2, Md, M48, T3, E):
    n_blk = P // BLK3
    return pl.pallas_call(
        _attn_kernel,
        grid=(n_blk,),
        in_specs=[
            pl.BlockSpec((BLK3, D), lambda i: (i, 0)),
            pl.BlockSpec((BLK3, GP), lambda i: (i, 0)),
            pl.BlockSpec((RB, D), lambda i: (i, 0)),
            pl.BlockSpec((RB, D), lambda i: (i, 0)),
            pl.BlockSpec((RB, GP), lambda i: (i, 0)),
            pl.BlockSpec((RB, 3), lambda i: (i, 0)),
            pl.BlockSpec((RB, H), lambda i: (i, 0)),
            pl.BlockSpec((GP, GP), lambda i: (0, 0)),
            pl.BlockSpec((1, GP), lambda i: (0, 0)),
            pl.BlockSpec((D, H), lambda i: (0, 0)),
            pl.BlockSpec((GP, H), lambda i: (0, 0)),
            pl.BlockSpec((3, GP), lambda i: (0, 0)),
            pl.BlockSpec((H, D), lambda i: (0, 0)),
        ],
        out_specs=pl.BlockSpec((BLK3, D), lambda i: (i, 0)),
        out_shape=jax.ShapeDtypeStruct((P, D), jnp.float32),
    )(q, bqv, kg, vg, ag, rpe_flat, rel_flat, Wg2, bg2.reshape(1, GP),
      Md, M48, T3, E)


def _masks():
    d = jnp.arange(D)
    h = jnp.arange(H)
    Md = (d[:, None] // HD == h[None, :]).astype(jnp.float32)
    j = jnp.arange(G)
    M48 = (j[:, None] // 3 == h[None, :]).astype(jnp.float32)
    M48 = jnp.pad(M48, ((0, GP - G), (0, 0)))
    perm = jnp.array([1, 2, 0])
    c = jnp.arange(3)
    T3 = (c[:, None] == perm[j % 3][None, :]).astype(jnp.float32)
    T3 = jnp.pad(T3, ((0, 0), (0, GP - G)))
    E = (d[None, :] // HD == h[:, None]).astype(jnp.float32)
    return Md, M48, T3, E


@jax.jit
def kernel(query, key, value, index_pair, query_batch_cnt, key_batch_cnt,
           index_pair_batch, relative_atten_weights, rpe_distance,
           Wq, bq, Wk, bk, Wv, bv, Wg1, bg1, Wg2, bg2):
    Wg1p = jnp.pad(Wg1, ((0, 0), (0, GP - G)))
    bg1p = jnp.pad(bg1, (0, GP - G))
    Wg2p = jnp.pad(Wg2, ((0, GP - G), (0, GP - G)))
    bg2p = jnp.pad(bg2, (0, GP - G))
    q, k, v, a, bqv = _stage1(query, key, value, Wq, bq, Wk, bk, Wv, bv,
                              Wg1p, bg1p)
    idx_flat = index_pair.reshape(R)
    kg, vg, ag = _stage2(k, v, a, idx_flat)
    Md, M48, T3, E = _masks()
    rpe_flat = rpe_distance.reshape(R, 3)
    rel_flat = relative_atten_weights.reshape(R, H)
    return _stage3(q, bqv, kg, vg, ag, rpe_flat, rel_flat, Wg2p, bg2p,
                   Md, M48, T3, E)
